# baseline (speedup 1.0000x reference)
import functools

import jax
import jax.numpy as jnp
from jax import lax
from jax.experimental import pallas as pl
from jax.experimental.pallas import tpu as pltpu
from jax.experimental.pallas import tpu_sc as plsc

_NC = 2
_NS = 16
_CH = 128
_NSB = 8


def _make_sc_agg(n_pad, e_pad, d, with_deg=False):
    nch = e_pad // (_NS * _CH)
    sbs = nch // _NSB
    rpt = n_pad // _NS
    half = d // 2

    mesh = plsc.VectorSubcoreMesh(core_axis_name="c", subcore_axis_name="s")

    out_type = [jax.ShapeDtypeStruct((_NC, n_pad, d), jnp.float32)]
    scratch = [
        pltpu.VMEM((sbs, _CH), jnp.int32),
        pltpu.VMEM((sbs, _CH), jnp.int32),
        pltpu.VMEM((2, _CH, half), jnp.float32),
        pltpu.VMEM_SHARED((n_pad, half), jnp.float32),
        pltpu.VMEM_SHARED((n_pad, half), jnp.float32),
        pltpu.SemaphoreType.DMA((2,)),
        pltpu.SemaphoreType.DMA((2,)),
    ]
    if with_deg:
        out_type.append(jax.ShapeDtypeStruct((_NC, n_pad, 16), jnp.float32))
        scratch.append(pltpu.VMEM((_CH, 16), jnp.float32))
        scratch.append(pltpu.VMEM_SHARED((n_pad, 16), jnp.float32))

    def body(h_hbm, hsw_hbm, srcm_hbm, dstm_hbm, zeros_hbm, *rest):
        if with_deg:
            (ones_hbm, out_hbm, deg_hbm,
             sidx_v, didx_v, rows_v, h_sh, agg_sh, gsems, ssems,
             ones_v, deg_sh) = rest
        else:
            (out_hbm, sidx_v, didx_v, rows_v, h_sh, agg_sh,
             gsems, ssems) = rest
        c = lax.axis_index("c")
        s = lax.axis_index("s")
        rs = pl.ds(s * rpt, rpt)

        @pl.when(c == 0)
        def _():
            pltpu.sync_copy(h_hbm.at[rs, pl.ds(0, half)], h_sh.at[rs])

        @pl.when(c == 1)
        def _():
            pltpu.sync_copy(hsw_hbm.at[rs, pl.ds(0, half)], h_sh.at[rs])

        pltpu.sync_copy(zeros_hbm.at[rs, pl.ds(0, half)], agg_sh.at[rs])
        if with_deg:
            pltpu.sync_copy(zeros_hbm.at[rs, pl.ds(0, 16)], deg_sh.at[rs])
            pltpu.sync_copy(ones_hbm, ones_v)
        plsc.subcore_barrier()

        def gather(j, b):
            pltpu.async_copy(h_sh.at[sidx_v.at[j]], rows_v.at[b],
                             gsems.at[b])

        def wait_scatter(j, b, deg_core=None):
            pltpu.make_async_copy(h_sh.at[sidx_v.at[j]], rows_v.at[b],
                                  gsems.at[b]).wait()
            pltpu.async_copy(rows_v.at[b], agg_sh.at[didx_v.at[j]],
                             ssems.at[b], add=True)
            if deg_core is not None:
                @pl.when(c == deg_core)
                def _():
                    pltpu.sync_copy(ones_v, deg_sh.at[didx_v.at[j]],
                                    add=True)

        def wait_scat_done(j, b):
            pltpu.make_async_copy(rows_v.at[b], agg_sh.at[didx_v.at[j]],
                                  ssems.at[b]).wait()

        for sb in range(_NSB):
            pltpu.sync_copy(srcm_hbm.at[s, sb], sidx_v)
            pltpu.sync_copy(dstm_hbm.at[s, sb], didx_v)
            deg_core = (sb % _NC) if with_deg else None

            gather(0, 0)
            gather(1, 1)

            def step(p, carry, deg_core=deg_core):
                j0 = 2 * p
                wait_scatter(j0, 0, deg_core)
                wait_scatter(j0 + 1, 1, deg_core)
                wait_scat_done(j0, 0)
                gather(j0 + 2, 0)
                wait_scat_done(j0 + 1, 1)
                gather(j0 + 3, 1)
                return carry

            lax.fori_loop(0, sbs // 2 - 1, step, 0)
            wait_scatter(sbs - 2, 0, deg_core)
            wait_scatter(sbs - 1, 1, deg_core)
            wait_scat_done(sbs - 2, 0)
            wait_scat_done(sbs - 1, 1)
        plsc.subcore_barrier()

        pltpu.sync_copy(agg_sh.at[rs],
                        out_hbm.at[c, rs, pl.ds(0, half)])
        if with_deg:
            pltpu.sync_copy(deg_sh.at[rs], deg_hbm.at[c, rs])

    return pl.kernel(body, out_type=out_type, mesh=mesh,
                     scratch_types=scratch,
                     compiler_params=pltpu.CompilerParams(
                         use_tc_tiling_on_sc=False))


def _tc_body(agg_ref, dinv_ref, h_ref, wl_ref, wr_ref, b_ref, *out_refs,
             relu, half, swap, first):
    if first:
        deg = dinv_ref[0][:, 0:1] + dinv_ref[1][:, 0:1]
        inv = 1.0 / jnp.maximum(deg, 1.0)
    else:
        inv = dinv_ref[...]
    mlo = agg_ref[0][:, :half] * inv
    mhi = agg_ref[1][:, :half] * inv
    acc = jnp.dot(mlo, wl_ref[...][:half, :],
                  preferred_element_type=jnp.float32)
    acc += jnp.dot(mhi, wl_ref[...][half:, :],
                   preferred_element_type=jnp.float32)
    acc += jnp.dot(h_ref[...], wr_ref[...], preferred_element_type=jnp.float32)
    acc += b_ref[...]
    if relu:
        acc = jnp.maximum(acc, 0.0)
    out_refs[0][...] = acc
    if swap:
        out_refs[1][...] = jnp.concatenate([acc[:, half:], acc[:, :half]],
                                           axis=1)
    if first:
        out_refs[2][...] = inv


def _tc_layer(aggp, dinv, h, wlt, wrt, b, bn, relu, swap, first=False):
    n_pad, d = h.shape
    grid = (n_pad // bn,)
    out_specs = [pl.BlockSpec((bn, d), lambda i: (i, 0))]
    out_shape = [jax.ShapeDtypeStruct((n_pad, d), jnp.float32)]
    if swap:
        out_specs.append(pl.BlockSpec((bn, d), lambda i: (i, 0)))
        out_shape.append(jax.ShapeDtypeStruct((n_pad, d), jnp.float32))
    if first:
        out_specs.append(pl.BlockSpec((bn, 1), lambda i: (i, 0)))
        out_shape.append(jax.ShapeDtypeStruct((n_pad, 1), jnp.float32))
    dinv_spec = (pl.BlockSpec((2, bn, 16), lambda i: (0, i, 0)) if first
                 else pl.BlockSpec((bn, 1), lambda i: (i, 0)))
    return pl.pallas_call(
        functools.partial(_tc_body, relu=relu, half=d // 2, swap=swap,
                          first=first),
        grid=grid,
        in_specs=[
            pl.BlockSpec((2, bn, d), lambda i: (0, i, 0)),
            dinv_spec,
            pl.BlockSpec((bn, d), lambda i: (i, 0)),
            pl.BlockSpec((d, d), lambda i: (0, 0)),
            pl.BlockSpec((d, d), lambda i: (0, 0)),
            pl.BlockSpec((1, d), lambda i: (0, 0)),
        ],
        out_specs=out_specs,
        out_shape=out_shape,
    )(aggp, dinv, h, wlt, wrt, b)


def kernel(x, edge_index, W1l, W1r, b1, W2l, W2r, b2, W3l, W3r, b3):
    n, d = x.shape
    e = edge_index.shape[1]
    half = d // 2
    n_pad = ((n + _NS * 8 - 1) // (_NS * 8)) * (_NS * 8)
    bn = n_pad // 8

    gran = _NS * _NSB * _CH
    e_pad = ((e + gran - 1) // gran) * gran
    nch = e_pad // (_NS * _CH)
    src = jnp.concatenate(
        [edge_index[0], jnp.zeros((e_pad - e,), edge_index.dtype)])
    dst = jnp.concatenate(
        [edge_index[1], jnp.full((e_pad - e,), n_pad - 1, edge_index.dtype)])
    srcm = src.reshape(_NS, _NSB, nch // _NSB, _CH)
    dstm = dst.reshape(_NS, _NSB, nch // _NSB, _CH)
    zeros = jnp.zeros((n_pad, d), jnp.float32)
    ones16 = jnp.ones((_CH, 16), jnp.float32)

    xp = jnp.zeros((n_pad, d), x.dtype).at[:n].set(x)
    xsw = jnp.concatenate([xp[:, half:], xp[:, :half]], axis=1)

    sc_agg = _make_sc_agg(n_pad, e_pad, d)
    sc_agg_deg = _make_sc_agg(n_pad, e_pad, d, with_deg=True)

    agg1, deg = sc_agg_deg(xp, xsw, srcm, dstm, zeros, ones16)
    h1, h1sw, inv = _tc_layer(agg1, deg, xp, W1l.T, W1r.T, b1[None, :], bn,
                              relu=True, swap=True, first=True)
    (agg2,) = sc_agg(h1, h1sw, srcm, dstm, zeros)
    h2, h2sw = _tc_layer(agg2, inv, h1, W2l.T, W2r.T, b2[None, :], bn,
                         relu=True, swap=True)
    (agg3,) = sc_agg(h2, h2sw, srcm, dstm, zeros)
    (outp,) = _tc_layer(agg3, inv, h2, W3l.T, W3r.T, b3[None, :], bn,
                        relu=False, swap=False)
    return outp[:n]


# speedup vs baseline: 1.1321x; 1.1321x over previous
import functools

import jax
import jax.numpy as jnp
from jax import lax
from jax.experimental import pallas as pl
from jax.experimental.pallas import tpu as pltpu
from jax.experimental.pallas import tpu_sc as plsc

_NC = 2
_NS = 16
_CH = 128
_NSB = 8


def _make_sc_agg(n_pad, e_pad, d, with_deg=False):
    nch = e_pad // (_NS * _CH)
    sbs = nch // _NSB
    rpt = n_pad // _NS
    half = d // 2

    mesh = plsc.VectorSubcoreMesh(core_axis_name="c", subcore_axis_name="s")

    out_type = [jax.ShapeDtypeStruct((_NC, n_pad, d), jnp.float32)]
    scratch = [
        pltpu.VMEM((sbs, _CH), jnp.int32),
        pltpu.VMEM((sbs, _CH), jnp.int32),
        pltpu.VMEM((2, _CH, half), jnp.float32),
        pltpu.VMEM_SHARED((n_pad, half), jnp.float32),
        pltpu.VMEM_SHARED((n_pad, half), jnp.float32),
        pltpu.SemaphoreType.DMA((2,)),
    ]
    if with_deg:
        out_type.append(jax.ShapeDtypeStruct((_NC, n_pad, 16), jnp.float32))
        scratch.append(pltpu.VMEM((_CH, 16), jnp.float32))
        scratch.append(pltpu.VMEM_SHARED((n_pad, 16), jnp.float32))

    def body(h_hbm, hsw_hbm, srcm_hbm, dstm_hbm, zeros_hbm, *rest):
        if with_deg:
            (ones_hbm, out_hbm, deg_hbm,
             sidx_v, didx_v, rows_v, h_sh, agg_sh, gsems,
             ones_v, deg_sh) = rest
        else:
            out_hbm, sidx_v, didx_v, rows_v, h_sh, agg_sh, gsems = rest
        c = lax.axis_index("c")
        s = lax.axis_index("s")
        rs = pl.ds(s * rpt, rpt)

        @pl.when(c == 0)
        def _():
            pltpu.sync_copy(h_hbm.at[rs, pl.ds(0, half)], h_sh.at[rs])

        @pl.when(c == 1)
        def _():
            pltpu.sync_copy(hsw_hbm.at[rs, pl.ds(0, half)], h_sh.at[rs])

        pltpu.sync_copy(zeros_hbm.at[rs, pl.ds(0, half)], agg_sh.at[rs])
        if with_deg:
            pltpu.sync_copy(zeros_hbm.at[rs, pl.ds(0, 16)], deg_sh.at[rs])
            pltpu.sync_copy(ones_hbm, ones_v)
        plsc.subcore_barrier()

        def gather(j, b):
            pltpu.async_copy(h_sh.at[sidx_v.at[j]], rows_v.at[b],
                             gsems.at[b])

        def wait_scatter(j, b, deg_core=None):
            pltpu.make_async_copy(h_sh.at[sidx_v.at[j]], rows_v.at[b],
                                  gsems.at[b]).wait()
            pltpu.sync_copy(rows_v.at[b], agg_sh.at[didx_v.at[j]], add=True)
            if deg_core is not None:
                @pl.when(c == deg_core)
                def _():
                    pltpu.sync_copy(ones_v, deg_sh.at[didx_v.at[j]],
                                    add=True)


        for sb in range(_NSB):
            pltpu.sync_copy(srcm_hbm.at[s, sb], sidx_v)
            pltpu.sync_copy(dstm_hbm.at[s, sb], didx_v)
            deg_core = (sb % _NC) if with_deg else None

            gather(0, 0)
            gather(1, 1)

            def step(p, carry, deg_core=deg_core):
                j0 = 2 * p
                wait_scatter(j0, 0, deg_core)
                gather(j0 + 2, 0)
                wait_scatter(j0 + 1, 1, deg_core)
                gather(j0 + 3, 1)
                return carry

            lax.fori_loop(0, sbs // 2 - 1, step, 0)
            wait_scatter(sbs - 2, 0, deg_core)
            wait_scatter(sbs - 1, 1, deg_core)
        plsc.subcore_barrier()

        pltpu.sync_copy(agg_sh.at[rs],
                        out_hbm.at[c, rs, pl.ds(0, half)])
        if with_deg:
            pltpu.sync_copy(deg_sh.at[rs], deg_hbm.at[c, rs])

    return pl.kernel(body, out_type=out_type, mesh=mesh,
                     scratch_types=scratch,
                     compiler_params=pltpu.CompilerParams(
                         use_tc_tiling_on_sc=False))


def _tc_body(agg_ref, dinv_ref, h_ref, wl_ref, wr_ref, b_ref, *out_refs,
             relu, half, swap, first):
    if first:
        deg = dinv_ref[0][:, 0:1] + dinv_ref[1][:, 0:1]
        inv = 1.0 / jnp.maximum(deg, 1.0)
    else:
        inv = dinv_ref[...]
    mlo = agg_ref[0][:, :half] * inv
    mhi = agg_ref[1][:, :half] * inv
    acc = jnp.dot(mlo, wl_ref[...][:half, :],
                  preferred_element_type=jnp.float32)
    acc += jnp.dot(mhi, wl_ref[...][half:, :],
                   preferred_element_type=jnp.float32)
    acc += jnp.dot(h_ref[...], wr_ref[...], preferred_element_type=jnp.float32)
    acc += b_ref[...]
    if relu:
        acc = jnp.maximum(acc, 0.0)
    out_refs[0][...] = acc
    if swap:
        out_refs[1][...] = jnp.concatenate([acc[:, half:], acc[:, :half]],
                                           axis=1)
    if first:
        out_refs[2][...] = inv


def _tc_layer(aggp, dinv, h, wlt, wrt, b, bn, relu, swap, first=False):
    n_pad, d = h.shape
    grid = (n_pad // bn,)
    out_specs = [pl.BlockSpec((bn, d), lambda i: (i, 0))]
    out_shape = [jax.ShapeDtypeStruct((n_pad, d), jnp.float32)]
    if swap:
        out_specs.append(pl.BlockSpec((bn, d), lambda i: (i, 0)))
        out_shape.append(jax.ShapeDtypeStruct((n_pad, d), jnp.float32))
    if first:
        out_specs.append(pl.BlockSpec((bn, 1), lambda i: (i, 0)))
        out_shape.append(jax.ShapeDtypeStruct((n_pad, 1), jnp.float32))
    dinv_spec = (pl.BlockSpec((2, bn, 16), lambda i: (0, i, 0)) if first
                 else pl.BlockSpec((bn, 1), lambda i: (i, 0)))
    return pl.pallas_call(
        functools.partial(_tc_body, relu=relu, half=d // 2, swap=swap,
                          first=first),
        grid=grid,
        in_specs=[
            pl.BlockSpec((2, bn, d), lambda i: (0, i, 0)),
            dinv_spec,
            pl.BlockSpec((bn, d), lambda i: (i, 0)),
            pl.BlockSpec((d, d), lambda i: (0, 0)),
            pl.BlockSpec((d, d), lambda i: (0, 0)),
            pl.BlockSpec((1, d), lambda i: (0, 0)),
        ],
        out_specs=out_specs,
        out_shape=out_shape,
    )(aggp, dinv, h, wlt, wrt, b)


def kernel(x, edge_index, W1l, W1r, b1, W2l, W2r, b2, W3l, W3r, b3):
    n, d = x.shape
    e = edge_index.shape[1]
    half = d // 2
    n_pad = ((n + _NS * 8 - 1) // (_NS * 8)) * (_NS * 8)
    bn = n_pad // 8

    gran = _NS * _NSB * _CH
    e_pad = ((e + gran - 1) // gran) * gran
    nch = e_pad // (_NS * _CH)
    src = jnp.concatenate(
        [edge_index[0], jnp.zeros((e_pad - e,), edge_index.dtype)])
    dst = jnp.concatenate(
        [edge_index[1], jnp.full((e_pad - e,), n_pad - 1, edge_index.dtype)])
    srcm = src.reshape(_NS, _NSB, nch // _NSB, _CH)
    dstm = dst.reshape(_NS, _NSB, nch // _NSB, _CH)
    zeros = jnp.zeros((n_pad, d), jnp.float32)
    ones16 = jnp.ones((_CH, 16), jnp.float32)

    xp = jnp.zeros((n_pad, d), x.dtype).at[:n].set(x)
    xsw = jnp.concatenate([xp[:, half:], xp[:, :half]], axis=1)

    sc_agg = _make_sc_agg(n_pad, e_pad, d)
    sc_agg_deg = _make_sc_agg(n_pad, e_pad, d, with_deg=True)

    agg1, deg = sc_agg_deg(xp, xsw, srcm, dstm, zeros, ones16)
    h1, h1sw, inv = _tc_layer(agg1, deg, xp, W1l.T, W1r.T, b1[None, :], bn,
                              relu=True, swap=True, first=True)
    (agg2,) = sc_agg(h1, h1sw, srcm, dstm, zeros)
    h2, h2sw = _tc_layer(agg2, inv, h1, W2l.T, W2r.T, b2[None, :], bn,
                         relu=True, swap=True)
    (agg3,) = sc_agg(h2, h2sw, srcm, dstm, zeros)
    (outp,) = _tc_layer(agg3, inv, h2, W3l.T, W3r.T, b3[None, :], bn,
                        relu=False, swap=False)
    return outp[:n]


# speedup vs baseline: 1.1496x; 1.0155x over previous
import functools

import jax
import jax.numpy as jnp
from jax import lax
from jax.experimental import pallas as pl
from jax.experimental.pallas import tpu as pltpu
from jax.experimental.pallas import tpu_sc as plsc

_NC = 2
_NS = 16
_CH = 128
_NSB = 8


def _make_sc_agg(n_pad, e_pad, d, with_deg=False):
    nch = e_pad // (_NS * _CH)
    sbs = nch // _NSB
    rpt = n_pad // _NS
    half = d // 2

    mesh = plsc.VectorSubcoreMesh(core_axis_name="c", subcore_axis_name="s")

    out_type = [jax.ShapeDtypeStruct((_NC, n_pad, d), jnp.float32)]
    scratch = [
        pltpu.VMEM((sbs, _CH), jnp.int32),
        pltpu.VMEM((sbs, _CH), jnp.int32),
        pltpu.VMEM((2, _CH, half), jnp.float32),
        pltpu.VMEM_SHARED((n_pad, half), jnp.float32),
        pltpu.VMEM_SHARED((n_pad, half), jnp.float32),
        pltpu.SemaphoreType.DMA((2,)),
    ]
    if with_deg:
        out_type.append(jax.ShapeDtypeStruct((_NC, n_pad, 16), jnp.float32))
        scratch.append(pltpu.VMEM((_CH, 16), jnp.float32))
        scratch.append(pltpu.VMEM_SHARED((n_pad, 16), jnp.float32))

    def body(h_hbm, srcm_hbm, dstm_hbm, zeros_hbm, *rest):
        if with_deg:
            (ones_hbm, out_hbm, deg_hbm,
             sidx_v, didx_v, rows_v, h_sh, agg_sh, gsems,
             ones_v, deg_sh) = rest
        else:
            out_hbm, sidx_v, didx_v, rows_v, h_sh, agg_sh, gsems = rest
        c = lax.axis_index("c")
        s = lax.axis_index("s")
        rs = pl.ds(s * rpt, rpt)

        @pl.when(c == 0)
        def _():
            pltpu.sync_copy(h_hbm.at[rs, pl.ds(0, half)], h_sh.at[rs])

        @pl.when(c == 1)
        def _():
            pltpu.sync_copy(h_hbm.at[rs, pl.ds(half, half)], h_sh.at[rs])

        pltpu.sync_copy(zeros_hbm.at[rs, pl.ds(0, half)], agg_sh.at[rs])
        if with_deg:
            pltpu.sync_copy(zeros_hbm.at[rs, pl.ds(0, 16)], deg_sh.at[rs])
            pltpu.sync_copy(ones_hbm, ones_v)
        plsc.subcore_barrier()

        def gather(j, b):
            pltpu.async_copy(h_sh.at[sidx_v.at[j]], rows_v.at[b],
                             gsems.at[b])

        def wait_scatter(j, b, deg_core=None):
            pltpu.make_async_copy(h_sh.at[sidx_v.at[j]], rows_v.at[b],
                                  gsems.at[b]).wait()
            pltpu.sync_copy(rows_v.at[b], agg_sh.at[didx_v.at[j]], add=True)
            if deg_core is not None:
                @pl.when(c == deg_core)
                def _():
                    pltpu.sync_copy(ones_v, deg_sh.at[didx_v.at[j]],
                                    add=True)


        for sb in range(_NSB):
            pltpu.sync_copy(srcm_hbm.at[s, sb], sidx_v)
            pltpu.sync_copy(dstm_hbm.at[s, sb], didx_v)
            deg_core = (sb % _NC) if with_deg else None

            gather(0, 0)
            gather(1, 1)

            def step(p, carry, deg_core=deg_core):
                j0 = 2 * p
                wait_scatter(j0, 0, deg_core)
                gather(j0 + 2, 0)
                wait_scatter(j0 + 1, 1, deg_core)
                gather(j0 + 3, 1)
                return carry

            lax.fori_loop(0, sbs // 2 - 1, step, 0)
            wait_scatter(sbs - 2, 0, deg_core)
            wait_scatter(sbs - 1, 1, deg_core)
        plsc.subcore_barrier()

        pltpu.sync_copy(agg_sh.at[rs],
                        out_hbm.at[c, rs, pl.ds(0, half)])
        if with_deg:
            pltpu.sync_copy(deg_sh.at[rs], deg_hbm.at[c, rs])

    return pl.kernel(body, out_type=out_type, mesh=mesh,
                     scratch_types=scratch,
                     compiler_params=pltpu.CompilerParams(
                         use_tc_tiling_on_sc=False))


def _tc_body(agg_ref, dinv_ref, h_ref, wl_ref, wr_ref, b_ref, *out_refs,
             relu, half, first):
    if first:
        deg = dinv_ref[0][:, 0:1] + dinv_ref[1][:, 0:1]
        inv = 1.0 / jnp.maximum(deg, 1.0)
    else:
        inv = dinv_ref[...]
    mlo = agg_ref[0][:, :half] * inv
    mhi = agg_ref[1][:, :half] * inv
    acc = jnp.dot(mlo, wl_ref[...][:half, :],
                  preferred_element_type=jnp.float32)
    acc += jnp.dot(mhi, wl_ref[...][half:, :],
                   preferred_element_type=jnp.float32)
    acc += jnp.dot(h_ref[...], wr_ref[...], preferred_element_type=jnp.float32)
    acc += b_ref[...]
    if relu:
        acc = jnp.maximum(acc, 0.0)
    out_refs[0][...] = acc
    if first:
        out_refs[1][...] = inv


def _tc_layer(aggp, dinv, h, wlt, wrt, b, bn, relu, first=False):
    n_pad, d = h.shape
    grid = (n_pad // bn,)
    out_specs = [pl.BlockSpec((bn, d), lambda i: (i, 0))]
    out_shape = [jax.ShapeDtypeStruct((n_pad, d), jnp.float32)]
    if first:
        out_specs.append(pl.BlockSpec((bn, 1), lambda i: (i, 0)))
        out_shape.append(jax.ShapeDtypeStruct((n_pad, 1), jnp.float32))
    dinv_spec = (pl.BlockSpec((2, bn, 16), lambda i: (0, i, 0)) if first
                 else pl.BlockSpec((bn, 1), lambda i: (i, 0)))
    return pl.pallas_call(
        functools.partial(_tc_body, relu=relu, half=d // 2, first=first),
        grid=grid,
        in_specs=[
            pl.BlockSpec((2, bn, d), lambda i: (0, i, 0)),
            dinv_spec,
            pl.BlockSpec((bn, d), lambda i: (i, 0)),
            pl.BlockSpec((d, d), lambda i: (0, 0)),
            pl.BlockSpec((d, d), lambda i: (0, 0)),
            pl.BlockSpec((1, d), lambda i: (0, 0)),
        ],
        out_specs=out_specs,
        out_shape=out_shape,
    )(aggp, dinv, h, wlt, wrt, b)


def kernel(x, edge_index, W1l, W1r, b1, W2l, W2r, b2, W3l, W3r, b3):
    n, d = x.shape
    e = edge_index.shape[1]
    half = d // 2
    n_pad = ((n + _NS * 8 - 1) // (_NS * 8)) * (_NS * 8)
    bn = n_pad // 8

    gran = _NS * _NSB * _CH
    e_pad = ((e + gran - 1) // gran) * gran
    nch = e_pad // (_NS * _CH)
    src = jnp.concatenate(
        [edge_index[0], jnp.zeros((e_pad - e,), edge_index.dtype)])
    dst = jnp.concatenate(
        [edge_index[1], jnp.full((e_pad - e,), n_pad - 1, edge_index.dtype)])
    srcm = src.reshape(_NS, _NSB, nch // _NSB, _CH)
    dstm = dst.reshape(_NS, _NSB, nch // _NSB, _CH)
    zeros = jnp.zeros((n_pad, d), jnp.float32)
    ones16 = jnp.ones((_CH, 16), jnp.float32)

    xp = jnp.zeros((n_pad, d), x.dtype).at[:n].set(x)

    sc_agg = _make_sc_agg(n_pad, e_pad, d)
    sc_agg_deg = _make_sc_agg(n_pad, e_pad, d, with_deg=True)

    agg1, deg = sc_agg_deg(xp, srcm, dstm, zeros, ones16)
    h1, inv = _tc_layer(agg1, deg, xp, W1l.T, W1r.T, b1[None, :], bn,
                        relu=True, first=True)
    (agg2,) = sc_agg(h1, srcm, dstm, zeros)
    (h2,) = _tc_layer(agg2, inv, h1, W2l.T, W2r.T, b2[None, :], bn,
                      relu=True)
    (agg3,) = sc_agg(h2, srcm, dstm, zeros)
    (outp,) = _tc_layer(agg3, inv, h2, W3l.T, W3r.T, b3[None, :], bn,
                        relu=False)
    return outp[:n]
